# Initial kernel scaffold; baseline (speedup 1.0000x reference)
import jax
import jax.numpy as jnp
from jax.experimental import pallas as pl


def kernel(Ea, E2a, Za, batch_seg):
    raise NotImplementedError("write your pallas kernel here")


# speedup vs baseline: 15.7137x; 15.7137x over previous
import jax
import jax.numpy as jnp
from jax import lax
from jax.experimental import pallas as pl
from jax.experimental.pallas import tpu as pltpu
from jax.experimental.pallas import tpu_sc as plsc

N = 3_200_000
NUM_SEG = 16_384
NC = 2
NS = 16
L = 16
NW = NC * NS
CHUNK = N // NW
SLICE = 10_000
NUM_SLICES = CHUNK // SLICE


def _sc_body(ea_hbm, e2a_hbm, seg_hbm, out_hbm, part_hbm,
             ea_v, e2a_v, seg_v, out_v, acc_v):
    wid = lax.axis_index("s") * NC + lax.axis_index("c")
    base = wid * CHUNK

    def zero_body(i, carry):
        acc_v[pl.ds(i * L, L)] = jnp.zeros((L,), jnp.float32)
        return carry
    lax.fori_loop(0, NUM_SEG // L, zero_body, 0)

    def slice_body(s, carry):
        off = base + s * SLICE
        pltpu.sync_copy(ea_hbm.at[pl.ds(off, SLICE)], ea_v)
        pltpu.sync_copy(e2a_hbm.at[pl.ds(off, SLICE)], e2a_v)
        pltpu.sync_copy(seg_hbm.at[pl.ds(off, SLICE)], seg_v)

        def vec_body(i, c):
            sl = pl.ds(i * L, L)
            ea = ea_v[sl]
            out_v[sl] = ea + e2a_v[sl]
            plsc.addupdate_scatter(acc_v, [seg_v[sl]], ea)
            return c
        lax.fori_loop(0, SLICE // L, vec_body, 0)

        pltpu.sync_copy(out_v, out_hbm.at[pl.ds(off, SLICE)])
        return carry
    lax.fori_loop(0, NUM_SLICES, slice_body, 0)

    pltpu.sync_copy(acc_v, part_hbm.at[wid])


def _combine_body(p_ref, e_ref):
    e_ref[...] = jnp.sum(p_ref[...], axis=0)


def kernel(Ea, E2a, Za, batch_seg):
    seg = batch_seg.astype(jnp.int32)
    mesh = plsc.VectorSubcoreMesh(core_axis_name="c", subcore_axis_name="s")
    sc = pl.kernel(
        _sc_body,
        out_type=(jax.ShapeDtypeStruct((N,), jnp.float32),
                  jax.ShapeDtypeStruct((NW, NUM_SEG), jnp.float32)),
        mesh=mesh,
        compiler_params=pltpu.CompilerParams(needs_layout_passes=False),
        scratch_types=[
            pltpu.VMEM((SLICE,), jnp.float32),
            pltpu.VMEM((SLICE,), jnp.float32),
            pltpu.VMEM((SLICE,), jnp.int32),
            pltpu.VMEM((SLICE,), jnp.float32),
            pltpu.VMEM((NUM_SEG,), jnp.float32),
        ],
    )
    ea_out, partials = sc(Ea, E2a, seg)
    e = pl.pallas_call(
        _combine_body,
        out_shape=jax.ShapeDtypeStruct((NUM_SEG,), jnp.float32),
    )(partials)
    return ea_out, e


# speedup vs baseline: 17.1669x; 1.0925x over previous
import jax
import jax.numpy as jnp
from jax import lax
from jax.experimental import pallas as pl
from jax.experimental.pallas import tpu as pltpu
from jax.experimental.pallas import tpu_sc as plsc

N = 3_200_000
NUM_SEG = 16_384
NC = 2
NS = 16
L = 16
NW = NC * NS
CHUNK = N // NW
SLICE = 10_000
NUM_SLICES = CHUNK // SLICE


VECS = SLICE // L
U_SC = 25
U_EW = 5


def _sc_body(ea_hbm, e2a_hbm, seg_hbm, out_hbm, part_hbm,
             ea_v, e2a_v, seg_v, out_v, acc_v):
    wid = lax.axis_index("s") * NC + lax.axis_index("c")
    base = wid * CHUNK

    def zero_body(i, carry):
        for j in range(16):
            acc_v[pl.ds((i * 16 + j) * L, L)] = jnp.zeros((L,), jnp.float32)
        return carry
    lax.fori_loop(0, NUM_SEG // (16 * L), zero_body, 0)

    def slice_body(s, carry):
        off = base + s * SLICE
        pltpu.sync_copy(ea_hbm.at[pl.ds(off, SLICE)], ea_v)
        pltpu.sync_copy(e2a_hbm.at[pl.ds(off, SLICE)], e2a_v)
        pltpu.sync_copy(seg_hbm.at[pl.ds(off, SLICE)], seg_v)

        @plsc.parallel_loop(0, VECS, 1, unroll=U_EW)
        def _(i):
            sl = pl.ds(i * L, L)
            out_v[sl] = ea_v[sl] + e2a_v[sl]

        def vec_body(i, c):
            for j in range(U_SC):
                sl = pl.ds((i * U_SC + j) * L, L)
                plsc.addupdate_scatter(acc_v, [seg_v[sl]], ea_v[sl])
            return c
        lax.fori_loop(0, VECS // U_SC, vec_body, 0)

        pltpu.sync_copy(out_v, out_hbm.at[pl.ds(off, SLICE)])
        return carry
    lax.fori_loop(0, NUM_SLICES, slice_body, 0)

    pltpu.sync_copy(acc_v, part_hbm.at[wid])


def _combine_body(p_ref, e_ref):
    e_ref[...] = jnp.sum(p_ref[...], axis=0)


def kernel(Ea, E2a, Za, batch_seg):
    seg = batch_seg.astype(jnp.int32)
    mesh = plsc.VectorSubcoreMesh(core_axis_name="c", subcore_axis_name="s")
    sc = pl.kernel(
        _sc_body,
        out_type=(jax.ShapeDtypeStruct((N,), jnp.float32),
                  jax.ShapeDtypeStruct((NW, NUM_SEG), jnp.float32)),
        mesh=mesh,
        compiler_params=pltpu.CompilerParams(needs_layout_passes=False),
        scratch_types=[
            pltpu.VMEM((SLICE,), jnp.float32),
            pltpu.VMEM((SLICE,), jnp.float32),
            pltpu.VMEM((SLICE,), jnp.int32),
            pltpu.VMEM((SLICE,), jnp.float32),
            pltpu.VMEM((NUM_SEG,), jnp.float32),
        ],
    )
    ea_out, partials = sc(Ea, E2a, seg)
    e = pl.pallas_call(
        _combine_body,
        out_shape=jax.ShapeDtypeStruct((NUM_SEG,), jnp.float32),
    )(partials)
    return ea_out, e


# speedup vs baseline: 20.2873x; 1.1818x over previous
import jax
import jax.numpy as jnp
from jax import lax
from jax.experimental import pallas as pl
from jax.experimental.pallas import tpu as pltpu
from jax.experimental.pallas import tpu_sc as plsc

N = 3_200_000
NUM_SEG = 16_384
NC = 2
NS = 16
L = 16
NW = NC * NS
CHUNK = N // NW
SLICE = 10_000
NUM_SLICES = CHUNK // SLICE


VECS = SLICE // L
U_SC = 25
U_EW = 5
NBUF = 2


def _sc_body(ea_hbm, e2a_hbm, seg_hbm, out_hbm, part_hbm,
             ea_v0, ea_v1, e2a_v0, e2a_v1, seg_v0, seg_v1, out_v0, out_v1,
             acc_v, in_sem0, in_sem1, out_sem0, out_sem1):
    wid = lax.axis_index("s") * NC + lax.axis_index("c")
    base = wid * CHUNK
    ea_bufs = (ea_v0, ea_v1)
    e2a_bufs = (e2a_v0, e2a_v1)
    seg_bufs = (seg_v0, seg_v1)
    out_bufs = (out_v0, out_v1)
    in_sems = (in_sem0, in_sem1)
    out_sems = (out_sem0, out_sem1)

    def zero_body(i, carry):
        for j in range(16):
            acc_v[pl.ds((i * 16 + j) * L, L)] = jnp.zeros((L,), jnp.float32)
        return carry
    lax.fori_loop(0, NUM_SEG // (16 * L), zero_body, 0)

    def start_in(s):
        off = base + s * SLICE
        b = s % NBUF
        return [
            pltpu.async_copy(ea_hbm.at[pl.ds(off, SLICE)], ea_bufs[b], in_sems[b]),
            pltpu.async_copy(e2a_hbm.at[pl.ds(off, SLICE)], e2a_bufs[b], in_sems[b]),
            pltpu.async_copy(seg_hbm.at[pl.ds(off, SLICE)], seg_bufs[b], in_sems[b]),
        ]

    pending_in = {0: start_in(0)}
    pending_out = {}
    for s in range(NUM_SLICES):
        b = s % NBUF
        if s + 1 < NUM_SLICES:
            pending_in[s + 1] = start_in(s + 1)
        for d in pending_in.pop(s):
            d.wait()
        if s - NBUF in pending_out:
            pending_out.pop(s - NBUF).wait()

        eab, e2ab, segb, outb = ea_bufs[b], e2a_bufs[b], seg_bufs[b], out_bufs[b]

        @plsc.parallel_loop(0, VECS, 1, unroll=U_EW)
        def _(i):
            sl = pl.ds(i * L, L)
            outb[sl] = eab[sl] + e2ab[sl]

        def vec_body(i, c):
            for j in range(U_SC):
                sl = pl.ds((i * U_SC + j) * L, L)
                plsc.addupdate_scatter(acc_v, [segb[sl]], eab[sl])
            return c
        lax.fori_loop(0, VECS // U_SC, vec_body, 0)

        off = base + s * SLICE
        pending_out[s] = pltpu.async_copy(
            outb, out_hbm.at[pl.ds(off, SLICE)], out_sems[b])
    for s in sorted(pending_out):
        pending_out[s].wait()

    pltpu.sync_copy(acc_v, part_hbm.at[wid])


def _combine_body(p_ref, e_ref):
    e_ref[...] = jnp.sum(p_ref[...], axis=0)


def kernel(Ea, E2a, Za, batch_seg):
    seg = batch_seg.astype(jnp.int32)
    mesh = plsc.VectorSubcoreMesh(core_axis_name="c", subcore_axis_name="s")
    sc = pl.kernel(
        _sc_body,
        out_type=(jax.ShapeDtypeStruct((N,), jnp.float32),
                  jax.ShapeDtypeStruct((NW, NUM_SEG), jnp.float32)),
        mesh=mesh,
        compiler_params=pltpu.CompilerParams(
            needs_layout_passes=False, disable_bounds_checks=True),
        scratch_types=[
            pltpu.VMEM((SLICE,), jnp.float32),
            pltpu.VMEM((SLICE,), jnp.float32),
            pltpu.VMEM((SLICE,), jnp.float32),
            pltpu.VMEM((SLICE,), jnp.float32),
            pltpu.VMEM((SLICE,), jnp.int32),
            pltpu.VMEM((SLICE,), jnp.int32),
            pltpu.VMEM((SLICE,), jnp.float32),
            pltpu.VMEM((SLICE,), jnp.float32),
            pltpu.VMEM((NUM_SEG,), jnp.float32),
            pltpu.SemaphoreType.DMA,
            pltpu.SemaphoreType.DMA,
            pltpu.SemaphoreType.DMA,
            pltpu.SemaphoreType.DMA,
        ],
    )
    ea_out, partials = sc(Ea, E2a, seg)
    e = pl.pallas_call(
        _combine_body,
        out_shape=jax.ShapeDtypeStruct((NUM_SEG,), jnp.float32),
    )(partials)
    return ea_out, e


# speedup vs baseline: 28.3400x; 1.3969x over previous
import jax
import jax.numpy as jnp
from jax import lax
from jax.experimental import pallas as pl
from jax.experimental.pallas import tpu as pltpu
from jax.experimental.pallas import tpu_sc as plsc

N = 3_200_000
NUM_SEG = 16_384
NC = 2
NS = 16
L = 16
NW = NC * NS
CHUNK = N // NW
SLICE = 10_000
NUM_SLICES = CHUNK // SLICE


VECS = SLICE // L
U_SC = 25
U_EW = 5
NBUF = 2


def _sc_body(ea_hbm, e2a_hbm, seg_hbm, out_hbm, part_hbm,
             ea_v0, ea_v1, e2a_v0, e2a_v1, seg_v0, seg_v1, out_v0, out_v1,
             acc_v, in_sem0, in_sem1, out_sem0, out_sem1):
    wid = lax.axis_index("s") * NC + lax.axis_index("c")
    base = wid * CHUNK
    ea_bufs = (ea_v0, ea_v1)
    e2a_bufs = (e2a_v0, e2a_v1)
    seg_bufs = (seg_v0, seg_v1)
    out_bufs = (out_v0, out_v1)
    in_sems = (in_sem0, in_sem1)
    out_sems = (out_sem0, out_sem1)

    def zero_body(i, carry):
        for j in range(16):
            acc_v[pl.ds((i * 16 + j) * L, L)] = jnp.zeros((L,), jnp.float32)
        return carry
    lax.fori_loop(0, NUM_SEG // (16 * L), zero_body, 0)

    def start_in(s):
        off = base + s * SLICE
        b = s % NBUF
        copies = [
            pltpu.async_copy(ea_hbm.at[pl.ds(off, SLICE)], ea_bufs[b], in_sems[b]),
            pltpu.async_copy(e2a_hbm.at[pl.ds(off, SLICE)], e2a_bufs[b], in_sems[b]),
            pltpu.async_copy(seg_hbm.at[pl.ds(off, SLICE)],
                             seg_bufs[b].at[pl.ds(0, SLICE)], in_sems[b]),
        ]
        if s + 1 < NUM_SLICES:
            copies.append(pltpu.async_copy(
                seg_hbm.at[pl.ds(off + SLICE, L)],
                seg_bufs[b].at[pl.ds(SLICE, L)], in_sems[b]))
        return copies

    pending_in = {0: start_in(0)}
    pending_out = {}
    run_vec = jnp.zeros((L,), jnp.float32)
    for s in range(NUM_SLICES):
        b = s % NBUF
        if s + 1 < NUM_SLICES:
            pending_in[s + 1] = start_in(s + 1)
        for d in pending_in.pop(s):
            d.wait()
        if s - NBUF in pending_out:
            pending_out.pop(s - NBUF).wait()

        eab, e2ab, segb, outb = ea_bufs[b], e2a_bufs[b], seg_bufs[b], out_bufs[b]
        if s == NUM_SLICES - 1:
            segb[pl.ds(SLICE, L)] = jnp.full((L,), -1, jnp.int32)

        def vec_body(i, rv):
            for j in range(U_SC):
                v = i * U_SC + j
                sl = pl.ds(v * L, L)
                ea = eab[sl]
                outb[sl] = ea + e2ab[sl]
                idx = segb[sl]
                nxt = segb[pl.ds(v * L + 1, L)]
                t = plsc.cumsum(ea) + rv
                end = idx != nxt
                plsc.addupdate_scatter(acc_v, [idx], t, mask=end)
                plsc.addupdate_scatter(acc_v, [nxt], -t,
                                       mask=end & (nxt >= 0))
                rv = rv + jnp.sum(ea)
            return rv
        run_vec = lax.fori_loop(0, VECS // U_SC, vec_body, run_vec)

        off = base + s * SLICE
        pending_out[s] = pltpu.async_copy(
            outb, out_hbm.at[pl.ds(off, SLICE)], out_sems[b])
    for s in sorted(pending_out):
        pending_out[s].wait()

    pltpu.sync_copy(acc_v, part_hbm.at[wid])


def _combine_body(p_ref, e_ref):
    e_ref[...] = jnp.sum(p_ref[...], axis=0)


def kernel(Ea, E2a, Za, batch_seg):
    seg = batch_seg.astype(jnp.int32)
    mesh = plsc.VectorSubcoreMesh(core_axis_name="c", subcore_axis_name="s")
    sc = pl.kernel(
        _sc_body,
        out_type=(jax.ShapeDtypeStruct((N,), jnp.float32),
                  jax.ShapeDtypeStruct((NW, NUM_SEG), jnp.float32)),
        mesh=mesh,
        compiler_params=pltpu.CompilerParams(
            needs_layout_passes=False, disable_bounds_checks=True),
        scratch_types=[
            pltpu.VMEM((SLICE,), jnp.float32),
            pltpu.VMEM((SLICE,), jnp.float32),
            pltpu.VMEM((SLICE,), jnp.float32),
            pltpu.VMEM((SLICE,), jnp.float32),
            pltpu.VMEM((SLICE + L,), jnp.int32),
            pltpu.VMEM((SLICE + L,), jnp.int32),
            pltpu.VMEM((SLICE,), jnp.float32),
            pltpu.VMEM((SLICE,), jnp.float32),
            pltpu.VMEM((NUM_SEG,), jnp.float32),
            pltpu.SemaphoreType.DMA,
            pltpu.SemaphoreType.DMA,
            pltpu.SemaphoreType.DMA,
            pltpu.SemaphoreType.DMA,
        ],
    )
    ea_out, partials = sc(Ea, E2a, seg)
    e = pl.pallas_call(
        _combine_body,
        out_shape=jax.ShapeDtypeStruct((NUM_SEG,), jnp.float32),
    )(partials)
    return ea_out, e
